# baseline (speedup 1.0000x reference)
import jax
import jax.numpy as jnp
from jax import lax
from jax.experimental import pallas as pl
from jax.experimental.pallas import tpu as pltpu
from jax.experimental.pallas import tpu_sc as plsc

ROWS, COLS = 16384, 1024
LANES = 16
NUM_WORKERS = 32
ROWS_PER_WORKER = ROWS // NUM_WORKERS
NBLK = ROWS_PER_WORKER // LANES
NBUF = 3
UNROLL = 4
SKEW = LANES - 1


def _cumsum_body(x_hbm, out_hbm, *refs):
    ibufs, obufs = refs[:NBUF], refs[NBUF:2 * NBUF]
    sems = refs[2 * NBUF:]
    in_sems, out_sems = sems[:NBUF], sems[NBUF:]
    wid = lax.axis_index("s") * 2 + lax.axis_index("c")
    r_base = wid * ROWS_PER_WORKER
    lane = lax.iota(jnp.int32, LANES)
    zero = jnp.zeros((LANES,), jnp.float32)

    def start_in(s):
        r0 = r_base + s * LANES
        return pltpu.async_copy(
            x_hbm.at[pl.ds(r0, LANES), :], ibufs[s % NBUF], in_sems[s % NBUF])

    def start_out(s):
        r0 = r_base + s * LANES
        return pltpu.async_copy(
            obufs[s % NBUF], out_hbm.at[pl.ds(r0, LANES), :], out_sems[s % NBUF])

    def edge_step(ibuf, obuf, acc, col, mask):
        c = jnp.clip(col, 0, COLS - 1)
        v = plsc.load_gather(ibuf, [lane, c], mask=mask)
        acc = acc + jnp.where(mask, v, 0.0)
        plsc.store_scatter(obuf, [lane, c], acc, mask=mask)
        return acc, col + 1

    in_descs = [None] * NBLK
    out_descs = [None] * NBLK
    for s in range(min(NBUF, NBLK)):
        in_descs[s] = start_in(s)

    for s in range(NBLK):
        in_descs[s].wait()
        ibuf, obuf = ibufs[s % NBUF], obufs[s % NBUF]

        acc, col = zero, -lane
        for j in range(SKEW):
            acc, col = edge_step(ibuf, obuf, acc, col, lane <= j)

        @plsc.parallel_loop(SKEW, COLS, unroll=UNROLL, carry=(acc, col))
        def steady(j, c, ibuf=ibuf, obuf=obuf):
            a, i = c
            a = a + plsc.load_gather(ibuf, [lane, i])
            plsc.store_scatter(obuf, [lane, i], a)
            return a, i + 1

        acc, col = steady
        for j in range(COLS, COLS + SKEW):
            acc, col = edge_step(ibuf, obuf, acc, col, lane >= j - (COLS - 1))

        if s >= NBUF:
            out_descs[s - NBUF].wait()
        out_descs[s] = start_out(s)
        if s + NBUF < NBLK:
            in_descs[s + NBUF] = start_in(s + NBUF)

    for s in range(NBLK - NBUF, NBLK):
        out_descs[s].wait()


_cumsum_sc = pl.kernel(
    _cumsum_body,
    out_type=jax.ShapeDtypeStruct((ROWS, COLS), jnp.float32),
    mesh=plsc.VectorSubcoreMesh(core_axis_name="c", subcore_axis_name="s"),
    scratch_types=(
        [pltpu.VMEM((LANES, COLS), jnp.float32) for _ in range(2 * NBUF)]
        + [pltpu.SemaphoreType.DMA for _ in range(2 * NBUF)]
    ),
    compiler_params=pltpu.CompilerParams(needs_layout_passes=False),
)


def kernel(x):
    return _cumsum_sc(x)


# speedup vs baseline: 1.0491x; 1.0491x over previous
import jax
import jax.numpy as jnp
from jax import lax
from jax.experimental import pallas as pl
from jax.experimental.pallas import tpu as pltpu
from jax.experimental.pallas import tpu_sc as plsc

ROWS, COLS = 16384, 1024
LANES = 16
NUM_WORKERS = 32
ROWS_PER_WORKER = ROWS // NUM_WORKERS
NBLK = ROWS_PER_WORKER // LANES
NBUF = 3
UNROLL = 8
SKEW = LANES - 1


def _cumsum_body(x_hbm, out_hbm, *refs):
    ibufs, obufs = refs[:NBUF], refs[NBUF:2 * NBUF]
    sems = refs[2 * NBUF:]
    in_sems, out_sems = sems[:NBUF], sems[NBUF:]
    wid = lax.axis_index("s") * 2 + lax.axis_index("c")
    r_base = wid * ROWS_PER_WORKER
    lane = lax.iota(jnp.int32, LANES)
    zero = jnp.zeros((LANES,), jnp.float32)

    def start_in(s):
        r0 = r_base + s * LANES
        return pltpu.async_copy(
            x_hbm.at[pl.ds(r0, LANES), :], ibufs[s % NBUF], in_sems[s % NBUF])

    def start_out(s):
        r0 = r_base + s * LANES
        return pltpu.async_copy(
            obufs[s % NBUF], out_hbm.at[pl.ds(r0, LANES), :], out_sems[s % NBUF])

    def edge_step(ibuf, obuf, acc, col, mask):
        c = jnp.clip(col, 0, COLS - 1)
        v = plsc.load_gather(ibuf, [lane, c], mask=mask)
        acc = acc + jnp.where(mask, v, 0.0)
        plsc.store_scatter(obuf, [lane, c], acc, mask=mask)
        return acc, col + 1

    in_descs = [None] * NBLK
    out_descs = [None] * NBLK
    for s in range(min(NBUF, NBLK)):
        in_descs[s] = start_in(s)

    for s in range(NBLK):
        in_descs[s].wait()
        ibuf, obuf = ibufs[s % NBUF], obufs[s % NBUF]

        acc, col = zero, -lane
        for j in range(SKEW):
            acc, col = edge_step(ibuf, obuf, acc, col, lane <= j)

        @plsc.parallel_loop(SKEW, COLS, unroll=UNROLL, carry=(acc, col))
        def steady(j, c, ibuf=ibuf, obuf=obuf):
            a, i = c
            a = a + plsc.load_gather(ibuf, [lane, i])
            plsc.store_scatter(obuf, [lane, i], a)
            return a, i + 1

        acc, col = steady
        for j in range(COLS, COLS + SKEW):
            acc, col = edge_step(ibuf, obuf, acc, col, lane >= j - (COLS - 1))

        if s >= NBUF:
            out_descs[s - NBUF].wait()
        out_descs[s] = start_out(s)
        if s + NBUF < NBLK:
            in_descs[s + NBUF] = start_in(s + NBUF)

    for s in range(NBLK - NBUF, NBLK):
        out_descs[s].wait()


_cumsum_sc = pl.kernel(
    _cumsum_body,
    out_type=jax.ShapeDtypeStruct((ROWS, COLS), jnp.float32),
    mesh=plsc.VectorSubcoreMesh(core_axis_name="c", subcore_axis_name="s"),
    scratch_types=(
        [pltpu.VMEM((LANES, COLS), jnp.float32) for _ in range(2 * NBUF)]
        + [pltpu.SemaphoreType.DMA for _ in range(2 * NBUF)]
    ),
    compiler_params=pltpu.CompilerParams(needs_layout_passes=False),
)


def kernel(x):
    return _cumsum_sc(x)
